# Initial kernel scaffold; baseline (speedup 1.0000x reference)
import functools

import jax
import jax.numpy as jnp
from jax import lax
from jax.experimental import pallas as pl
from jax.experimental.pallas import tpu as pltpu
from jax.experimental.pallas import tpu_sc as plsc

D = 2048
L = 24
N = 576
T = 2048
RED = 512
FOCUS = 22
FINAL_FOCUS = 32
FINAL_NONFOCUS = 32
THRESH = 3.0
MIN_ACTIVE = 8
EPS = 1e-6
LN_EPS = 1e-5
NEG = -3.0e38

TEXT_CHUNKS = 8


def _text_body(t_ref, tg_ref, acc_ref):
    i = pl.program_id(0)

    @pl.when(i == 0)
    def _init():
        acc_ref[...] = jnp.zeros_like(acc_ref)

    acc_ref[...] += jnp.sum(t_ref[...], axis=0, keepdims=True)

    @pl.when(i == pl.num_programs(0) - 1)
    def _fin():
        mean = acc_ref[...] / T
        mu = jnp.mean(mean, axis=1, keepdims=True)
        var = jnp.mean((mean - mu) ** 2, axis=1, keepdims=True)
        tg_ref[...] = (mean - mu) / jnp.sqrt(var + LN_EPS)


def _means_body(x_ref, y_ref):
    y_ref[0, 0, :] = jnp.mean(x_ref[0], axis=0)


def _scan_body(y_ref, tg_ref, w1_ref, b1_ref, wc_ref, wi_ref, wf_ref,
               bc_ref, bi_ref, bf_ref, ctx_ref, yw_ref):
    y = y_ref[:, 0, :]
    tg = tg_ref[...]
    w1a = w1_ref[0:D, :]
    w1y = w1_ref[D:2 * D, :]
    w1t = w1_ref[2 * D:3 * D, :]
    yw_ref[...] = jnp.dot(y, w1y, preferred_element_type=jnp.float32)
    base = (jnp.dot(tg, w1t, preferred_element_type=jnp.float32)
            + b1_ref[...])
    wc = wc_ref[...]
    wi = wi_ref[...]
    wf = wf_ref[...]
    bcv = bc_ref[...]
    biv = bi_ref[...]
    bfv = bf_ref[...]

    def step(l, c):
        cn = jax.nn.sigmoid(c)
        h = (jnp.dot(cn, w1a, preferred_element_type=jnp.float32)
             + yw_ref[pl.ds(l, 1), :] + base)
        s = jnp.maximum(h, 0.0)
        ct = jnp.tanh(jnp.dot(s, wc, preferred_element_type=jnp.float32) + bcv)
        ig = jax.nn.sigmoid(jnp.dot(s, wi, preferred_element_type=jnp.float32) + biv)
        fg = jax.nn.sigmoid(jnp.dot(s, wf, preferred_element_type=jnp.float32) + bfv)
        c = fg * c + ig * ct
        ctx_ref[pl.ds(l, 1)] = c[:, None, :]
        return c

    lax.fori_loop(0, L, step, jnp.zeros((1, D), jnp.float32))


def _q_body(ctx_ref, wq_ref, g_ref, b_ref, q_ref):
    ctx = ctx_ref[:, 0, :]
    qr = jnp.dot(ctx, wq_ref[...], preferred_element_type=jnp.float32)
    mu = jnp.mean(qr, axis=1, keepdims=True)
    var = jnp.mean((qr - mu) ** 2, axis=1, keepdims=True)
    qn = (qr - mu) / jnp.sqrt(var + LN_EPS)
    q = qn * g_ref[...] + b_ref[...]
    q_ref[...] = q[:, None, :]


def _big_body(x_ref, wk_ref, q_ref, g_ref, b_ref, sval_ref, sidx_ref):
    x = x_ref[0]
    ql = q_ref[0]
    a = ql * g_ref[...]
    c_add = jnp.sum(b_ref[...] * ql)
    v = jnp.dot(x, wk_ref[...], preferred_element_type=jnp.float32)
    m = jnp.mean(v, axis=1, keepdims=True)
    vc = v - m
    var = jnp.mean(vc * vc, axis=1, keepdims=True)
    dot = jnp.sum(vc * a, axis=1, keepdims=True)
    raw = dot / jnp.sqrt(var + LN_EPS) + c_add
    mu = jnp.mean(raw)
    sd = jnp.sqrt(jnp.mean((raw - mu) ** 2))
    sc = (raw - mu) / (sd + EPS)

    iota_n = lax.broadcasted_iota(jnp.int32, (N, 1), 0)
    iota_k = lax.broadcasted_iota(jnp.int32, (1, FINAL_NONFOCUS), 1)

    def step(j, carry):
        s, vals, idxs = carry
        mx = jnp.max(s)
        idx = jnp.min(jnp.where(s == mx, iota_n, N))
        vals = jnp.where(iota_k == j, mx, vals)
        idxs = jnp.where(iota_k == j, idx, idxs)
        s = jnp.where(iota_n == idx, NEG, s)
        return s, vals, idxs

    _, vals, idxs = lax.fori_loop(
        0, FINAL_NONFOCUS, step,
        (sc,
         jnp.zeros((1, FINAL_NONFOCUS), jnp.float32),
         jnp.zeros((1, FINAL_NONFOCUS), jnp.int32)))
    sval_ref[...] = vals[None]
    sidx_ref[...] = idxs[None]


def _sel_body(sval_ref, sidx_ref, flat_ref, w_ref):
    sv = sval_ref[:, 0, :]
    si = sidx_ref[:, 0, :]
    K = FINAL_NONFOCUS

    ri = lax.broadcasted_iota(jnp.int32, (L, L), 0)
    ci = lax.broadcasted_iota(jnp.int32, (L, L), 1)

    def col2row(v):
        return jnp.sum(jnp.where(ri == ci, v, 0 * v), axis=0, keepdims=True)

    def cumsum_to_col(v_row):
        return jnp.sum(jnp.where(ci <= ri, v_row, 0 * v_row),
                       axis=1, keepdims=True)

    conf = col2row(sv[:, 0:1])
    li = lax.broadcasted_iota(jnp.int32, (1, L), 1)
    nonfocus = li != FOCUS
    active = nonfocus & (conf > THRESH)
    act_i = active.astype(jnp.int32)
    m0 = jnp.sum(act_i)
    rank_a = col2row(cumsum_to_col(act_i)) - 1
    base_a = K // jnp.maximum(m0, 1)
    rem_a = K - base_a * m0
    k_a = jnp.where(active,
                    jnp.minimum(base_a + (rank_a < rem_a).astype(jnp.int32), N),
                    0)
    key_b = jnp.where(nonfocus, -conf, 3.0e38)
    kb_col = jnp.sum(jnp.where(ri == ci, key_b, 0.0 * key_b),
                     axis=1, keepdims=True)
    less = (key_b < kb_col).astype(jnp.int32)
    eq_before = ((key_b == kb_col) & (ci < ri)).astype(jnp.int32)
    rank_b = col2row(jnp.sum(less + eq_before, axis=1, keepdims=True))
    mb = min(MIN_ACTIVE, L - 1)
    base_b = K // mb
    rem_b = K - base_b * mb
    k_b = jnp.where(rank_b < mb,
                    jnp.minimum(base_b + (rank_b < rem_b).astype(jnp.int32), N),
                    0)
    k_per = jnp.where(m0 >= MIN_ACTIVE, k_a, k_b)
    cum = col2row(cumsum_to_col(k_per))
    cumex = cum - k_per

    s_col = lax.broadcasted_iota(jnp.int32, (K, 1), 0)
    lay = jnp.sum((cum <= s_col).astype(jnp.int32), axis=1, keepdims=True)
    li_row = lax.broadcasted_iota(jnp.int32, (1, L), 1)
    oh = (lay == li_row).astype(jnp.float32)
    cumex_s = jnp.sum(oh * cumex.astype(jnp.float32), axis=1,
                      keepdims=True).astype(jnp.int32)
    pos = s_col - cumex_s

    rowvals = jnp.dot(oh, sv, preferred_element_type=jnp.float32)
    rowidx = jnp.dot(oh, si.astype(jnp.float32),
                     preferred_element_type=jnp.float32)
    pi = lax.broadcasted_iota(jnp.int32, (1, K), 1)
    pmask = (pos == pi).astype(jnp.float32)
    sel_score = jnp.sum(rowvals * pmask, axis=1, keepdims=True)
    sel_idx = jnp.sum(rowidx * pmask, axis=1, keepdims=True).astype(jnp.int32)
    sel_lay = lay

    rif = lax.broadcasted_iota(jnp.int32, (K, K), 0)
    cif = lax.broadcasted_iota(jnp.int32, (K, K), 1)
    fscore = jnp.sum(jnp.where(rif == cif, sv[FOCUS:FOCUS + 1, :], 0.0),
                     axis=1, keepdims=True)
    fidx = jnp.sum(jnp.where(rif == cif,
                             si[FOCUS:FOCUS + 1, :].astype(jnp.float32), 0.0),
                   axis=1, keepdims=True).astype(jnp.int32)

    all_score = jnp.concatenate([fscore, sel_score], axis=0)
    mxs = jnp.max(all_score)
    e = jnp.exp(all_score - mxs)
    w = e / jnp.sum(e)
    all_flat = jnp.concatenate(
        [FOCUS * N + fidx, sel_lay * N + sel_idx], axis=0)
    flat_ref[...] = all_flat
    w_ref[...] = w


def _wmul_body(rows_ref, w_ref, out_ref):
    out_ref[...] = rows_ref[...] * w_ref[...]


_SC_NC = 2
_SC_NS = 16
_GW = 8


def _sc_gather_body(table_hbm, idx_hbm, out_hbm, idx_v, rows_v, sem):
    wid = lax.axis_index("s") * _SC_NC + lax.axis_index("c")

    @pl.when(wid < _GW)
    def _():
        base = wid * ((FINAL_FOCUS + FINAL_NONFOCUS) // _GW)
        pltpu.sync_copy(idx_hbm.at[pl.ds(base, 8)], idx_v)
        pltpu.async_copy(table_hbm.at[idx_v], rows_v, sem).wait()
        pltpu.sync_copy(rows_v, out_hbm.at[pl.ds(base, 8)])


def _sc_gather(table, idx):
    mesh = plsc.VectorSubcoreMesh(core_axis_name="c", subcore_axis_name="s")
    kfn = functools.partial(
        pl.kernel,
        mesh=mesh,
        out_type=jax.ShapeDtypeStruct((FINAL_FOCUS + FINAL_NONFOCUS, D),
                                      jnp.float32),
        scratch_types=[
            pltpu.VMEM((8,), jnp.int32),
            pltpu.VMEM((8, D), jnp.float32),
            pltpu.SemaphoreType.DMA,
        ],
    )(_sc_gather_body)
    return kfn(table, idx)


def kernel(text_features, layer_feats, W1_w, W1_b, Wc_w, Wc_b, Wi_w, Wi_b,
           Wf_w, Wf_b, bc, bi, bf, Wq, Wk, ln_g, ln_b):
    f32 = jnp.float32
    g2 = ln_g.reshape(1, D)
    b2 = ln_b.reshape(1, D)

    tg = pl.pallas_call(
        _text_body,
        grid=(TEXT_CHUNKS,),
        in_specs=[pl.BlockSpec((T // TEXT_CHUNKS, D), lambda i: (i, 0))],
        out_specs=pl.BlockSpec((1, D), lambda i: (0, 0)),
        out_shape=jax.ShapeDtypeStruct((1, D), f32),
        scratch_shapes=[pltpu.VMEM((1, D), f32)],
    )(text_features)

    y3 = pl.pallas_call(
        _means_body,
        grid=(L,),
        in_specs=[pl.BlockSpec((1, N, D), lambda l: (l, 0, 0))],
        out_specs=pl.BlockSpec((1, 1, D), lambda l: (l, 0, 0)),
        out_shape=jax.ShapeDtypeStruct((L, 1, D), f32),
    )(layer_feats)

    ctx3 = pl.pallas_call(
        _scan_body,
        out_shape=jax.ShapeDtypeStruct((L, 1, D), f32),
        scratch_shapes=[pltpu.VMEM((L, RED), f32)],
    )(y3, tg, W1_w, W1_b.reshape(1, RED), Wc_w, Wi_w, Wf_w,
      (Wc_b + bc).reshape(1, D), (Wi_b + bi).reshape(1, D),
      (Wf_b + bf).reshape(1, D))

    q3 = pl.pallas_call(
        _q_body,
        out_shape=jax.ShapeDtypeStruct((L, 1, D), f32),
    )(ctx3, Wq, g2, b2)

    sval3, sidx3 = pl.pallas_call(
        _big_body,
        grid=(L,),
        in_specs=[
            pl.BlockSpec((1, N, D), lambda l: (l, 0, 0)),
            pl.BlockSpec((D, D), lambda l: (0, 0)),
            pl.BlockSpec((1, 1, D), lambda l: (l, 0, 0)),
            pl.BlockSpec((1, D), lambda l: (0, 0)),
            pl.BlockSpec((1, D), lambda l: (0, 0)),
        ],
        out_specs=[
            pl.BlockSpec((1, 1, FINAL_NONFOCUS), lambda l: (l, 0, 0)),
            pl.BlockSpec((1, 1, FINAL_NONFOCUS), lambda l: (l, 0, 0)),
        ],
        out_shape=[
            jax.ShapeDtypeStruct((L, 1, FINAL_NONFOCUS), f32),
            jax.ShapeDtypeStruct((L, 1, FINAL_NONFOCUS), jnp.int32),
        ],
    )(layer_feats, Wk, q3, g2, b2)

    flatc, wcol = pl.pallas_call(
        _sel_body,
        out_shape=[
            jax.ShapeDtypeStruct((FINAL_FOCUS + FINAL_NONFOCUS, 1), jnp.int32),
            jax.ShapeDtypeStruct((FINAL_FOCUS + FINAL_NONFOCUS, 1), f32),
        ],
    )(sval3, sidx3)

    table = layer_feats.reshape(L * N, D)
    rows = _sc_gather(table, flatc.reshape(FINAL_FOCUS + FINAL_NONFOCUS))

    out = pl.pallas_call(
        _wmul_body,
        out_shape=jax.ShapeDtypeStruct((FINAL_FOCUS + FINAL_NONFOCUS, D), f32),
    )(rows, wcol)
    return out


# speedup vs baseline: 1.0020x; 1.0020x over previous
import functools

import jax
import jax.numpy as jnp
from jax import lax
from jax.experimental import pallas as pl
from jax.experimental.pallas import tpu as pltpu
from jax.experimental.pallas import tpu_sc as plsc

D = 2048
L = 24
N = 576
T = 2048
RED = 512
FOCUS = 22
FINAL_FOCUS = 32
FINAL_NONFOCUS = 32
THRESH = 3.0
MIN_ACTIVE = 8
EPS = 1e-6
LN_EPS = 1e-5
NEG = -3.0e38

TEXT_CHUNKS = 8


def _text_body(t_ref, tg_ref, acc_ref):
    i = pl.program_id(0)

    @pl.when(i == 0)
    def _init():
        acc_ref[...] = jnp.zeros_like(acc_ref)

    acc_ref[...] += jnp.sum(t_ref[...], axis=0, keepdims=True)

    @pl.when(i == pl.num_programs(0) - 1)
    def _fin():
        mean = acc_ref[...] / T
        mu = jnp.mean(mean, axis=1, keepdims=True)
        var = jnp.mean((mean - mu) ** 2, axis=1, keepdims=True)
        tg_ref[...] = (mean - mu) / jnp.sqrt(var + LN_EPS)


def _means_body(x_ref, y_ref):
    y_ref[0, 0, :] = jnp.mean(x_ref[0], axis=0)


def _scan_body(y_ref, tg_ref, w1_ref, b1_ref, wc_ref, wi_ref, wf_ref,
               bc_ref, bi_ref, bf_ref, ctx_ref):
    bf16 = jnp.bfloat16
    tgb = tg_ref[...].astype(bf16)
    w1 = w1_ref[...]
    wc = wc_ref[...]
    wi = wi_ref[...]
    wf = wf_ref[...]
    bcv = bc_ref[...]
    biv = bi_ref[...]
    bfv = bf_ref[...]
    b1v = b1_ref[...]

    def step(l, c):
        cn = jax.nn.sigmoid(c)
        yl = y_ref[pl.ds(l, 1), 0, :]
        comb = jnp.concatenate(
            [cn.astype(bf16), yl.astype(bf16), tgb], axis=1)
        h = jnp.dot(comb, w1, preferred_element_type=jnp.float32) + b1v
        s = jnp.maximum(h, 0.0).astype(bf16)
        ct = jnp.tanh(jnp.dot(s, wc, preferred_element_type=jnp.float32) + bcv)
        ig = jax.nn.sigmoid(jnp.dot(s, wi, preferred_element_type=jnp.float32) + biv)
        fg = jax.nn.sigmoid(jnp.dot(s, wf, preferred_element_type=jnp.float32) + bfv)
        c = fg * c + ig * ct
        ctx_ref[pl.ds(l, 1)] = c[:, None, :]
        return c

    lax.fori_loop(0, L, step, jnp.zeros((1, D), jnp.float32))


def _q_body(ctx_ref, wq_ref, g_ref, b_ref, q_ref):
    ctx = ctx_ref[:, 0, :]
    qr = jnp.dot(ctx.astype(jnp.bfloat16), wq_ref[...],
                 preferred_element_type=jnp.float32)
    mu = jnp.mean(qr, axis=1, keepdims=True)
    var = jnp.mean((qr - mu) ** 2, axis=1, keepdims=True)
    qn = (qr - mu) / jnp.sqrt(var + LN_EPS)
    q = qn * g_ref[...] + b_ref[...]
    q_ref[...] = q[:, None, :]


def _big_body(x_ref, wk_ref, q_ref, g_ref, b_ref, sval_ref, sidx_ref):
    x = x_ref[0]
    ql = q_ref[0]
    a = ql * g_ref[...]
    c_add = jnp.sum(b_ref[...] * ql)
    v = jnp.dot(x.astype(jnp.bfloat16), wk_ref[...],
                preferred_element_type=jnp.float32)
    m = jnp.mean(v, axis=1, keepdims=True)
    vc = v - m
    var = jnp.mean(vc * vc, axis=1, keepdims=True)
    dot = jnp.sum(vc * a, axis=1, keepdims=True)
    raw = dot / jnp.sqrt(var + LN_EPS) + c_add
    mu = jnp.mean(raw)
    sd = jnp.sqrt(jnp.mean((raw - mu) ** 2))
    sc = (raw - mu) / (sd + EPS)

    iota_n = lax.broadcasted_iota(jnp.int32, (N, 1), 0)
    iota_k = lax.broadcasted_iota(jnp.int32, (1, FINAL_NONFOCUS), 1)

    def step(j, carry):
        s, vals, idxs = carry
        mx = jnp.max(s)
        idx = jnp.min(jnp.where(s == mx, iota_n, N))
        vals = jnp.where(iota_k == j, mx, vals)
        idxs = jnp.where(iota_k == j, idx, idxs)
        s = jnp.where(iota_n == idx, NEG, s)
        return s, vals, idxs

    _, vals, idxs = lax.fori_loop(
        0, FINAL_NONFOCUS, step,
        (sc,
         jnp.zeros((1, FINAL_NONFOCUS), jnp.float32),
         jnp.zeros((1, FINAL_NONFOCUS), jnp.int32)))
    sval_ref[...] = vals[None]
    sidx_ref[...] = idxs[None]


def _sel_body(sval_ref, sidx_ref, flat_ref, w_ref):
    sv = sval_ref[:, 0, :]
    si = sidx_ref[:, 0, :]
    K = FINAL_NONFOCUS

    ri = lax.broadcasted_iota(jnp.int32, (L, L), 0)
    ci = lax.broadcasted_iota(jnp.int32, (L, L), 1)

    def col2row(v):
        return jnp.sum(jnp.where(ri == ci, v, 0 * v), axis=0, keepdims=True)

    def cumsum_to_col(v_row):
        return jnp.sum(jnp.where(ci <= ri, v_row, 0 * v_row),
                       axis=1, keepdims=True)

    conf = col2row(sv[:, 0:1])
    li = lax.broadcasted_iota(jnp.int32, (1, L), 1)
    nonfocus = li != FOCUS
    active = nonfocus & (conf > THRESH)
    act_i = active.astype(jnp.int32)
    m0 = jnp.sum(act_i)
    rank_a = col2row(cumsum_to_col(act_i)) - 1
    base_a = K // jnp.maximum(m0, 1)
    rem_a = K - base_a * m0
    k_a = jnp.where(active,
                    jnp.minimum(base_a + (rank_a < rem_a).astype(jnp.int32), N),
                    0)
    key_b = jnp.where(nonfocus, -conf, 3.0e38)
    kb_col = jnp.sum(jnp.where(ri == ci, key_b, 0.0 * key_b),
                     axis=1, keepdims=True)
    less = (key_b < kb_col).astype(jnp.int32)
    eq_before = ((key_b == kb_col) & (ci < ri)).astype(jnp.int32)
    rank_b = col2row(jnp.sum(less + eq_before, axis=1, keepdims=True))
    mb = min(MIN_ACTIVE, L - 1)
    base_b = K // mb
    rem_b = K - base_b * mb
    k_b = jnp.where(rank_b < mb,
                    jnp.minimum(base_b + (rank_b < rem_b).astype(jnp.int32), N),
                    0)
    k_per = jnp.where(m0 >= MIN_ACTIVE, k_a, k_b)
    cum = col2row(cumsum_to_col(k_per))
    cumex = cum - k_per

    s_col = lax.broadcasted_iota(jnp.int32, (K, 1), 0)
    lay = jnp.sum((cum <= s_col).astype(jnp.int32), axis=1, keepdims=True)
    li_row = lax.broadcasted_iota(jnp.int32, (1, L), 1)
    oh = (lay == li_row).astype(jnp.float32)
    cumex_s = jnp.sum(oh * cumex.astype(jnp.float32), axis=1,
                      keepdims=True).astype(jnp.int32)
    pos = s_col - cumex_s

    rowvals = jnp.zeros((K, FINAL_NONFOCUS), jnp.float32)
    rowidx = jnp.zeros((K, FINAL_NONFOCUS), jnp.int32)
    for l in range(L):
        rowvals = jnp.where(lay == l, sv[l:l + 1, :], rowvals)
        rowidx = jnp.where(lay == l, si[l:l + 1, :], rowidx)
    pi = lax.broadcasted_iota(jnp.int32, (1, K), 1)
    pmask = pos == pi
    sel_score = jnp.sum(jnp.where(pmask, rowvals, 0.0), axis=1, keepdims=True)
    sel_idx = jnp.sum(jnp.where(pmask, rowidx, 0), axis=1, keepdims=True)
    sel_lay = lay

    rif = lax.broadcasted_iota(jnp.int32, (K, K), 0)
    cif = lax.broadcasted_iota(jnp.int32, (K, K), 1)
    fscore = jnp.sum(jnp.where(rif == cif, sv[FOCUS:FOCUS + 1, :], 0.0),
                     axis=1, keepdims=True)
    fidx = jnp.sum(jnp.where(rif == cif,
                             si[FOCUS:FOCUS + 1, :].astype(jnp.float32), 0.0),
                   axis=1, keepdims=True).astype(jnp.int32)

    all_score = jnp.concatenate([fscore, sel_score], axis=0)
    mxs = jnp.max(all_score)
    e = jnp.exp(all_score - mxs)
    w = e / jnp.sum(e)
    all_flat = jnp.concatenate(
        [FOCUS * N + fidx, sel_lay * N + sel_idx], axis=0)
    flat_ref[...] = all_flat
    w_ref[...] = w


def _wmul_body(rows_ref, w_ref, out_ref):
    out_ref[...] = rows_ref[...] * w_ref[...]


_SC_NC = 2
_SC_NS = 16
_GW = 8


def _sc_gather_body(table_hbm, idx_hbm, out_hbm, idx_v, rows_v, sem):
    wid = lax.axis_index("s") * _SC_NC + lax.axis_index("c")

    @pl.when(wid < _GW)
    def _():
        base = wid * ((FINAL_FOCUS + FINAL_NONFOCUS) // _GW)
        pltpu.sync_copy(idx_hbm.at[pl.ds(base, 8)], idx_v)
        pltpu.async_copy(table_hbm.at[idx_v], rows_v, sem).wait()
        pltpu.sync_copy(rows_v, out_hbm.at[pl.ds(base, 8)])


def _sc_gather(table, idx):
    mesh = plsc.VectorSubcoreMesh(core_axis_name="c", subcore_axis_name="s")
    kfn = functools.partial(
        pl.kernel,
        mesh=mesh,
        out_type=jax.ShapeDtypeStruct((FINAL_FOCUS + FINAL_NONFOCUS, D),
                                      jnp.float32),
        scratch_types=[
            pltpu.VMEM((8,), jnp.int32),
            pltpu.VMEM((8, D), jnp.float32),
            pltpu.SemaphoreType.DMA,
        ],
    )(_sc_gather_body)
    return kfn(table, idx)


def _stages(text_features, layer_feats, W1_w, W1_b, Wc_w, Wc_b, Wi_w, Wi_b,
            Wf_w, Wf_b, bc, bi, bf, Wq, Wk, ln_g, ln_b):
    f32 = jnp.float32
    g2 = ln_g.reshape(1, D)
    b2 = ln_b.reshape(1, D)

    tg = pl.pallas_call(
        _text_body,
        grid=(TEXT_CHUNKS,),
        in_specs=[pl.BlockSpec((T // TEXT_CHUNKS, D), lambda i: (i, 0))],
        out_specs=pl.BlockSpec((1, D), lambda i: (0, 0)),
        out_shape=jax.ShapeDtypeStruct((1, D), f32),
        scratch_shapes=[pltpu.VMEM((1, D), f32)],
    )(text_features)

    y3 = pl.pallas_call(
        _means_body,
        grid=(L,),
        in_specs=[pl.BlockSpec((1, N, D), lambda l: (l, 0, 0))],
        out_specs=pl.BlockSpec((1, 1, D), lambda l: (l, 0, 0)),
        out_shape=jax.ShapeDtypeStruct((L, 1, D), f32),
    )(layer_feats)

    ctx3 = pl.pallas_call(
        _scan_body,
        out_shape=jax.ShapeDtypeStruct((L, 1, D), f32),
    )(y3, tg, W1_w.astype(jnp.bfloat16), W1_b.reshape(1, RED),
      Wc_w.astype(jnp.bfloat16), Wi_w.astype(jnp.bfloat16),
      Wf_w.astype(jnp.bfloat16),
      (Wc_b + bc).reshape(1, D), (Wi_b + bi).reshape(1, D),
      (Wf_b + bf).reshape(1, D))

    q3 = pl.pallas_call(
        _q_body,
        out_shape=jax.ShapeDtypeStruct((L, 1, D), f32),
    )(ctx3, Wq.astype(jnp.bfloat16), g2, b2)

    sval3, sidx3 = pl.pallas_call(
        _big_body,
        grid=(L,),
        in_specs=[
            pl.BlockSpec((1, N, D), lambda l: (l, 0, 0)),
            pl.BlockSpec((D, D), lambda l: (0, 0)),
            pl.BlockSpec((1, 1, D), lambda l: (l, 0, 0)),
            pl.BlockSpec((1, D), lambda l: (0, 0)),
            pl.BlockSpec((1, D), lambda l: (0, 0)),
        ],
        out_specs=[
            pl.BlockSpec((1, 1, FINAL_NONFOCUS), lambda l: (l, 0, 0)),
            pl.BlockSpec((1, 1, FINAL_NONFOCUS), lambda l: (l, 0, 0)),
        ],
        out_shape=[
            jax.ShapeDtypeStruct((L, 1, FINAL_NONFOCUS), f32),
            jax.ShapeDtypeStruct((L, 1, FINAL_NONFOCUS), jnp.int32),
        ],
    )(layer_feats, Wk.astype(jnp.bfloat16), q3, g2, b2)

    flatc, wcol = pl.pallas_call(
        _sel_body,
        out_shape=[
            jax.ShapeDtypeStruct((FINAL_FOCUS + FINAL_NONFOCUS, 1), jnp.int32),
            jax.ShapeDtypeStruct((FINAL_FOCUS + FINAL_NONFOCUS, 1), f32),
        ],
    )(sval3, sidx3)
    return tg, y3, ctx3, q3, sval3, sidx3, flatc, wcol


def kernel(text_features, layer_feats, W1_w, W1_b, Wc_w, Wc_b, Wi_w, Wi_b,
           Wf_w, Wf_b, bc, bi, bf, Wq, Wk, ln_g, ln_b):
    f32 = jnp.float32
    (tg, y3, ctx3, q3, sval3, sidx3, flatc, wcol) = _stages(
        text_features, layer_feats, W1_w, W1_b, Wc_w, Wc_b, Wi_w, Wi_b,
        Wf_w, Wf_b, bc, bi, bf, Wq, Wk, ln_g, ln_b)

    table = layer_feats.reshape(L * N, D)
    rows = _sc_gather(table, flatc.reshape(FINAL_FOCUS + FINAL_NONFOCUS))

    out = pl.pallas_call(
        _wmul_body,
        out_shape=jax.ShapeDtypeStruct((FINAL_FOCUS + FINAL_NONFOCUS, D), f32),
    )(rows, wcol)
    return out
